# baseline (speedup 1.0000x reference)
import functools

import jax
import jax.numpy as jnp
from jax import lax
from jax.experimental import pallas as pl
from jax.experimental.pallas import tpu as pltpu
from jax.experimental.pallas import tpu_sc as plsc

N_CELL, N_TISSUE, D = 10000, 512, 128
NP = 10240
E_CELL, E_TISSUE = 320000, 4096
NC, NS = 2, 16
NW = NC * NS
CH = 128
NCHUNK = 80
EPW = NCHUNK * CH
EP = NW * EPW
RPT = NP // NS
PAD_ROW = NP - 1

_sc_mesh = plsc.VectorSubcoreMesh(core_axis_name="c", subcore_axis_name="s")


@functools.partial(
    pl.kernel,
    mesh=_sc_mesh,
    out_type=jax.ShapeDtypeStruct((NC, NP, D), jnp.float32),
    scratch_types=[
        pltpu.VMEM((CH,), jnp.int32),
        pltpu.VMEM((CH,), jnp.int32),
        pltpu.VMEM((CH, D), jnp.float32),
        pltpu.VMEM_SHARED((NP, D), jnp.float32),
        pltpu.SemaphoreType.DMA,
    ],
)
def _segsum_cell(src_hbm, dst_hbm, h_hbm, zeros_hbm, out_hbm,
                 src_v, dst_v, rows_v, acc_sh, sem):
    c = lax.axis_index("c")
    s = lax.axis_index("s")
    wid = s * NC + c
    pltpu.sync_copy(zeros_hbm, acc_sh.at[pl.ds(s * RPT, RPT)])
    plsc.subcore_barrier()
    base = wid * EPW

    def body(k, carry):
        off = pl.multiple_of(base + k * CH, 8)
        pltpu.sync_copy(src_hbm.at[pl.ds(off, CH)], src_v)
        pltpu.sync_copy(dst_hbm.at[pl.ds(off, CH)], dst_v)
        pltpu.async_copy(h_hbm.at[src_v], rows_v, sem).wait()
        pltpu.sync_copy(rows_v, acc_sh.at[dst_v], add=True)
        return carry

    lax.fori_loop(0, NCHUNK, body, 0)
    plsc.subcore_barrier()
    pltpu.sync_copy(acc_sh.at[pl.ds(s * RPT, RPT)],
                    out_hbm.at[c, pl.ds(s * RPT, RPT)])


def _gin_body(h_ref, p_ref, w1_ref, b1_ref, w2_ref, b2_ref, o_ref):
    z = h_ref[...] + p_ref[0] + p_ref[1]
    u = jnp.maximum(
        jnp.dot(z, w1_ref[...], preferred_element_type=jnp.float32) + b1_ref[...], 0.0)
    o_ref[...] = jnp.maximum(
        jnp.dot(u, w2_ref[...], preferred_element_type=jnp.float32) + b2_ref[...], 0.0)


def _gin_tc(h, p, W1, b1, W2, b2):
    RB = 2048
    return pl.pallas_call(
        _gin_body,
        grid=(NP // RB,),
        in_specs=[
            pl.BlockSpec((RB, D), lambda i: (i, 0)),
            pl.BlockSpec((NC, RB, D), lambda i: (0, i, 0)),
            pl.BlockSpec((D, D), lambda i: (0, 0)),
            pl.BlockSpec((1, D), lambda i: (0, 0)),
            pl.BlockSpec((D, D), lambda i: (0, 0)),
            pl.BlockSpec((1, D), lambda i: (0, 0)),
        ],
        out_specs=pl.BlockSpec((RB, D), lambda i: (i, 0)),
        out_shape=jax.ShapeDtypeStruct((NP, D), jnp.float32),
    )(h, p, W1, b1.reshape(1, D), W2, b2.reshape(1, D))


EB = 1024


def _at_body(s_ref, d_ref, o_ref):
    i = pl.program_id(0)
    se = s_ref[0, 0, :]
    de = d_ref[0, 0, :]
    cols = lax.broadcasted_iota(jnp.int32, (EB, N_TISSUE), 1)
    ohs = (se[:, None] == cols).astype(jnp.float32)
    ohd = (de[:, None] == cols).astype(jnp.float32)
    prod = lax.dot_general(ohd, ohs, (((0,), (0,)), ((), ())),
                           preferred_element_type=jnp.float32)

    @pl.when(i == 0)
    def _():
        o_ref[...] = prod

    @pl.when(i != 0)
    def _():
        o_ref[...] += prod


def _build_at(src3, dst3):
    return pl.pallas_call(
        _at_body,
        grid=(E_TISSUE // EB,),
        in_specs=[
            pl.BlockSpec((1, 1, EB), lambda i: (i, 0, 0)),
            pl.BlockSpec((1, 1, EB), lambda i: (i, 0, 0)),
        ],
        out_specs=pl.BlockSpec((N_TISSUE, N_TISSUE), lambda i: (0, 0)),
        out_shape=jax.ShapeDtypeStruct((N_TISSUE, N_TISSUE), jnp.float32),
    )(src3, dst3)


RB2 = 1000
NG2 = N_CELL // RB2


def _tail_body(h1_ref, h2_ref, h3_ref, a_ref, xt_ref, at_ref,
               t1w1, t1b1, t1w2, t1b2, t2w1, t2b1, t2w2, t2b2,
               t3w1, t3b1, t3w2, t3b2, cw1, cb1, cw2, cb2,
               o_ref, zc_ref):
    i = pl.program_id(0)
    a = a_ref[0, 0, :]
    z = jnp.concatenate([h1_ref[...], h2_ref[...], h3_ref[...]], axis=1)
    cols = lax.broadcasted_iota(jnp.int32, (RB2, N_TISSUE), 1)
    oh = (a[:, None] == cols).astype(jnp.float32)
    part = lax.dot_general(oh, z, (((0,), (0,)), ((), ())),
                           preferred_element_type=jnp.float32)

    @pl.when(i == 0)
    def _():
        zc_ref[...] = part

    @pl.when(i != 0)
    def _():
        zc_ref[...] += part

    @pl.when(i == NG2 - 1)
    def _():
        ht = jnp.concatenate([zc_ref[...], xt_ref[...]], axis=1)
        A = at_ref[...]
        touts = []
        for (w1, b1, w2, b2) in ((t1w1, t1b1, t1w2, t1b2),
                                 (t2w1, t2b1, t2w2, t2b2),
                                 (t3w1, t3b1, t3w2, t3b2)):
            agg = jnp.dot(A, ht, preferred_element_type=jnp.float32)
            zt = ht + agg
            u = jnp.maximum(
                jnp.dot(zt, w1[...], preferred_element_type=jnp.float32) + b1[...], 0.0)
            ht = jnp.maximum(
                jnp.dot(u, w2[...], preferred_element_type=jnp.float32) + b2[...], 0.0)
            touts.append(ht)
        ztc = jnp.concatenate(touts, axis=1)
        g = jnp.sum(ztc, axis=0, keepdims=True) * (1.0 / N_TISSUE)
        hc = jnp.maximum(
            jnp.dot(g, cw1[...], preferred_element_type=jnp.float32) + cb1[...], 0.0)
        o_ref[...] = jnp.dot(hc, cw2[...], preferred_element_type=jnp.float32) + cb2[...]


def _tail(h1, h2, h3, asg3, x_tissue, At, tw, cls_W1, cls_b1, cls_W2p, cls_b2p):
    def full(shape):
        nzero = len(shape)
        return pl.BlockSpec(shape, lambda i, _n=nzero: (0,) * _n)

    in_specs = [
        pl.BlockSpec((RB2, D), lambda i: (i, 0)),
        pl.BlockSpec((RB2, D), lambda i: (i, 0)),
        pl.BlockSpec((RB2, D), lambda i: (i, 0)),
        pl.BlockSpec((1, 1, RB2), lambda i: (i, 0, 0)),
        full((N_TISSUE, D)),
        full((N_TISSUE, N_TISSUE)),
    ]
    args = [h1, h2, h3, asg3, x_tissue, At]
    for (W1, b1, W2, b2) in tw:
        in_specs += [full(W1.shape), full((1, D)), full(W2.shape), full((1, D))]
        args += [W1, b1.reshape(1, D), W2, b2.reshape(1, D)]
    in_specs += [full(cls_W1.shape), full((1, D)),
                 full(cls_W2p.shape), full((1, D))]
    args += [cls_W1, cls_b1.reshape(1, D), cls_W2p, cls_b2p]
    return pl.pallas_call(
        _tail_body,
        grid=(NG2,),
        in_specs=in_specs,
        out_specs=pl.BlockSpec((1, D), lambda i: (0, 0)),
        out_shape=jax.ShapeDtypeStruct((1, D), jnp.float32),
        scratch_shapes=[pltpu.VMEM((N_TISSUE, 3 * D), jnp.float32)],
    )(*args)


def kernel(x_cell, x_tissue, edge_index_cell, edge_index_tissue, assignment,
           x_cell_batch, x_tissue_batch,
           c1_W1, c1_b1, c1_W2, c1_b2, c2_W1, c2_b1, c2_W2, c2_b2,
           c3_W1, c3_b1, c3_W2, c3_b2,
           t1_W1, t1_b1, t1_W2, t1_b2, t2_W1, t2_b1, t2_W2, t2_b2,
           t3_W1, t3_b1, t3_W2, t3_b2,
           cls_W1, cls_b1, cls_W2, cls_b2):
    npad = EP - E_CELL
    src_c = jnp.pad(edge_index_cell[0], (0, npad), constant_values=PAD_ROW)
    dst_c = jnp.pad(edge_index_cell[1], (0, npad), constant_values=PAD_ROW)
    zeros = jnp.zeros((RPT, D), jnp.float32)

    h = jnp.zeros((NP, D), jnp.float32).at[:N_CELL].set(x_cell)
    hs = []
    for (W1, b1, W2, b2) in ((c1_W1, c1_b1, c1_W2, c1_b2),
                             (c2_W1, c2_b1, c2_W2, c2_b2),
                             (c3_W1, c3_b1, c3_W2, c3_b2)):
        p = _segsum_cell(src_c, dst_c, h, zeros)
        h = _gin_tc(h, p, W1, b1, W2, b2)
        hs.append(h)

    At = _build_at(edge_index_tissue[0].reshape(E_TISSUE // EB, 1, EB),
                   edge_index_tissue[1].reshape(E_TISSUE // EB, 1, EB))

    cls_W2p = jnp.zeros((D, D), jnp.float32).at[:, :7].set(cls_W2)
    cls_b2p = jnp.zeros((1, D), jnp.float32).at[:, :7].set(cls_b2)

    out = _tail(hs[0], hs[1], hs[2], assignment.reshape(NG2, 1, RB2),
                x_tissue, At,
                ((t1_W1, t1_b1, t1_W2, t1_b2),
                 (t2_W1, t2_b1, t2_W2, t2_b2),
                 (t3_W1, t3_b1, t3_W2, t3_b2)),
                cls_W1, cls_b1, cls_W2p, cls_b2p)
    return out[:, :7]


# speedup vs baseline: 1.9524x; 1.9524x over previous
import functools

import jax
import jax.numpy as jnp
from jax import lax
from jax.experimental import pallas as pl
from jax.experimental.pallas import tpu as pltpu
from jax.experimental.pallas import tpu_sc as plsc

N_CELL, N_TISSUE, D = 10000, 512, 128
NP = 10240
E_CELL, E_TISSUE = 320000, 4096
NC, NS = 2, 16
NW = NC * NS
CH = 80
NCHUNK = 128
EPW = NCHUNK * CH
EP = NW * EPW
RPT = NP // NS
PAD_ROW = NP - 1

_sc_mesh = plsc.VectorSubcoreMesh(core_axis_name="c", subcore_axis_name="s")


@functools.partial(
    pl.kernel,
    mesh=_sc_mesh,
    out_type=jax.ShapeDtypeStruct((NC, NP, D), jnp.float32),
    scratch_types=[
        pltpu.VMEM((CH,), jnp.int32),
        pltpu.VMEM((CH,), jnp.int32),
        pltpu.VMEM((CH, D), jnp.float32),
        pltpu.VMEM_SHARED((NP, D), jnp.float32),
        pltpu.SemaphoreType.DMA,
    ],
)
def _segsum_cell(src_hbm, dst_hbm, h_hbm, zeros_hbm, out_hbm,
                 src_v, dst_v, rows_v, acc_sh, sem):
    c = lax.axis_index("c")
    s = lax.axis_index("s")
    wid = s * NC + c
    pltpu.sync_copy(zeros_hbm, acc_sh.at[pl.ds(s * RPT, RPT)])
    plsc.subcore_barrier()
    base = wid * EPW

    def body(k, carry):
        off = pl.multiple_of(base + k * CH, 8)
        pltpu.sync_copy(src_hbm.at[pl.ds(off, CH)], src_v)
        pltpu.sync_copy(dst_hbm.at[pl.ds(off, CH)], dst_v)
        pltpu.async_copy(h_hbm.at[src_v], rows_v, sem).wait()
        pltpu.sync_copy(rows_v, acc_sh.at[dst_v], add=True)
        return carry

    lax.fori_loop(0, NCHUNK, body, 0)
    plsc.subcore_barrier()
    pltpu.sync_copy(acc_sh.at[pl.ds(s * RPT, RPT)],
                    out_hbm.at[c, pl.ds(s * RPT, RPT)])


def _gin_body(h_ref, p_ref, w1_ref, b1_ref, w2_ref, b2_ref, o_ref):
    z = h_ref[...] + p_ref[0] + p_ref[1]
    u = jnp.maximum(
        jnp.dot(z, w1_ref[...], preferred_element_type=jnp.float32) + b1_ref[...], 0.0)
    o_ref[...] = jnp.maximum(
        jnp.dot(u, w2_ref[...], preferred_element_type=jnp.float32) + b2_ref[...], 0.0)


def _gin_tc(h, p, W1, b1, W2, b2):
    RB = 2048
    return pl.pallas_call(
        _gin_body,
        grid=(NP // RB,),
        in_specs=[
            pl.BlockSpec((RB, D), lambda i: (i, 0)),
            pl.BlockSpec((NC, RB, D), lambda i: (0, i, 0)),
            pl.BlockSpec((D, D), lambda i: (0, 0)),
            pl.BlockSpec((1, D), lambda i: (0, 0)),
            pl.BlockSpec((D, D), lambda i: (0, 0)),
            pl.BlockSpec((1, D), lambda i: (0, 0)),
        ],
        out_specs=pl.BlockSpec((RB, D), lambda i: (i, 0)),
        out_shape=jax.ShapeDtypeStruct((NP, D), jnp.float32),
    )(h, p, W1, b1.reshape(1, D), W2, b2.reshape(1, D))


EB = 1024


def _at_body(s_ref, d_ref, o_ref):
    i = pl.program_id(0)
    se = s_ref[0, 0, :]
    de = d_ref[0, 0, :]
    cols = lax.broadcasted_iota(jnp.int32, (EB, N_TISSUE), 1)
    ohs = (se[:, None] == cols).astype(jnp.float32)
    ohd = (de[:, None] == cols).astype(jnp.float32)
    prod = lax.dot_general(ohd, ohs, (((0,), (0,)), ((), ())),
                           preferred_element_type=jnp.float32)

    @pl.when(i == 0)
    def _():
        o_ref[...] = prod

    @pl.when(i != 0)
    def _():
        o_ref[...] += prod


def _build_at(src3, dst3):
    return pl.pallas_call(
        _at_body,
        grid=(E_TISSUE // EB,),
        in_specs=[
            pl.BlockSpec((1, 1, EB), lambda i: (i, 0, 0)),
            pl.BlockSpec((1, 1, EB), lambda i: (i, 0, 0)),
        ],
        out_specs=pl.BlockSpec((N_TISSUE, N_TISSUE), lambda i: (0, 0)),
        out_shape=jax.ShapeDtypeStruct((N_TISSUE, N_TISSUE), jnp.float32),
    )(src3, dst3)


RB2 = 1000
NG2 = N_CELL // RB2


def _tail_body(h1_ref, h2_ref, h3_ref, a_ref, xt_ref, at_ref,
               t1w1, t1b1, t1w2, t1b2, t2w1, t2b1, t2w2, t2b2,
               t3w1, t3b1, t3w2, t3b2, cw1, cb1, cw2, cb2,
               o_ref, zc_ref):
    i = pl.program_id(0)
    a = a_ref[0, 0, :]
    z = jnp.concatenate([h1_ref[...], h2_ref[...], h3_ref[...]], axis=1)
    cols = lax.broadcasted_iota(jnp.int32, (RB2, N_TISSUE), 1)
    oh = (a[:, None] == cols).astype(jnp.float32)
    part = lax.dot_general(oh, z, (((0,), (0,)), ((), ())),
                           preferred_element_type=jnp.float32)

    @pl.when(i == 0)
    def _():
        zc_ref[...] = part

    @pl.when(i != 0)
    def _():
        zc_ref[...] += part

    @pl.when(i == NG2 - 1)
    def _():
        ht = jnp.concatenate([zc_ref[...], xt_ref[...]], axis=1)
        A = at_ref[...]
        touts = []
        for (w1, b1, w2, b2) in ((t1w1, t1b1, t1w2, t1b2),
                                 (t2w1, t2b1, t2w2, t2b2),
                                 (t3w1, t3b1, t3w2, t3b2)):
            agg = jnp.dot(A, ht, preferred_element_type=jnp.float32)
            zt = ht + agg
            u = jnp.maximum(
                jnp.dot(zt, w1[...], preferred_element_type=jnp.float32) + b1[...], 0.0)
            ht = jnp.maximum(
                jnp.dot(u, w2[...], preferred_element_type=jnp.float32) + b2[...], 0.0)
            touts.append(ht)
        ztc = jnp.concatenate(touts, axis=1)
        g = jnp.sum(ztc, axis=0, keepdims=True) * (1.0 / N_TISSUE)
        hc = jnp.maximum(
            jnp.dot(g, cw1[...], preferred_element_type=jnp.float32) + cb1[...], 0.0)
        o_ref[...] = jnp.dot(hc, cw2[...], preferred_element_type=jnp.float32) + cb2[...]


def _tail(h1, h2, h3, asg3, x_tissue, At, tw, cls_W1, cls_b1, cls_W2p, cls_b2p):
    def full(shape):
        nzero = len(shape)
        return pl.BlockSpec(shape, lambda i, _n=nzero: (0,) * _n)

    in_specs = [
        pl.BlockSpec((RB2, D), lambda i: (i, 0)),
        pl.BlockSpec((RB2, D), lambda i: (i, 0)),
        pl.BlockSpec((RB2, D), lambda i: (i, 0)),
        pl.BlockSpec((1, 1, RB2), lambda i: (i, 0, 0)),
        full((N_TISSUE, D)),
        full((N_TISSUE, N_TISSUE)),
    ]
    args = [h1, h2, h3, asg3, x_tissue, At]
    for (W1, b1, W2, b2) in tw:
        in_specs += [full(W1.shape), full((1, D)), full(W2.shape), full((1, D))]
        args += [W1, b1.reshape(1, D), W2, b2.reshape(1, D)]
    in_specs += [full(cls_W1.shape), full((1, D)),
                 full(cls_W2p.shape), full((1, D))]
    args += [cls_W1, cls_b1.reshape(1, D), cls_W2p, cls_b2p]
    return pl.pallas_call(
        _tail_body,
        grid=(NG2,),
        in_specs=in_specs,
        out_specs=pl.BlockSpec((1, D), lambda i: (0, 0)),
        out_shape=jax.ShapeDtypeStruct((1, D), jnp.float32),
        scratch_shapes=[pltpu.VMEM((N_TISSUE, 3 * D), jnp.float32)],
    )(*args)


def kernel(x_cell, x_tissue, edge_index_cell, edge_index_tissue, assignment,
           x_cell_batch, x_tissue_batch,
           c1_W1, c1_b1, c1_W2, c1_b2, c2_W1, c2_b1, c2_W2, c2_b2,
           c3_W1, c3_b1, c3_W2, c3_b2,
           t1_W1, t1_b1, t1_W2, t1_b2, t2_W1, t2_b1, t2_W2, t2_b2,
           t3_W1, t3_b1, t3_W2, t3_b2,
           cls_W1, cls_b1, cls_W2, cls_b2):
    npad = EP - E_CELL
    pad_rows = N_CELL + (jnp.arange(npad, dtype=jnp.int32) % (NP - N_CELL))
    src_c = jnp.concatenate([edge_index_cell[0], pad_rows])
    dst_c = jnp.concatenate([edge_index_cell[1], pad_rows])
    zeros = jnp.zeros((RPT, D), jnp.float32)

    h = jnp.zeros((NP, D), jnp.float32).at[:N_CELL].set(x_cell)
    hs = []
    for (W1, b1, W2, b2) in ((c1_W1, c1_b1, c1_W2, c1_b2),
                             (c2_W1, c2_b1, c2_W2, c2_b2),
                             (c3_W1, c3_b1, c3_W2, c3_b2)):
        p = _segsum_cell(src_c, dst_c, h, zeros)
        h = _gin_tc(h, p, W1, b1, W2, b2)
        hs.append(h)

    At = _build_at(edge_index_tissue[0].reshape(E_TISSUE // EB, 1, EB),
                   edge_index_tissue[1].reshape(E_TISSUE // EB, 1, EB))

    cls_W2p = jnp.zeros((D, D), jnp.float32).at[:, :7].set(cls_W2)
    cls_b2p = jnp.zeros((1, D), jnp.float32).at[:, :7].set(cls_b2)

    out = _tail(hs[0], hs[1], hs[2], assignment.reshape(NG2, 1, RB2),
                x_tissue, At,
                ((t1_W1, t1_b1, t1_W2, t1_b2),
                 (t2_W1, t2_b1, t2_W2, t2_b2),
                 (t3_W1, t3_b1, t3_W2, t3_b2)),
                cls_W1, cls_b1, cls_W2p, cls_b2p)
    return out[:, :7]


# speedup vs baseline: 3.7412x; 1.9162x over previous
import functools

import jax
import jax.numpy as jnp
from jax import lax
from jax.experimental import pallas as pl
from jax.experimental.pallas import tpu as pltpu
from jax.experimental.pallas import tpu_sc as plsc

N_CELL, N_TISSUE, D = 10000, 512, 128
NP = 10240
E_CELL, E_TISSUE = 320000, 4096
NC, NS = 2, 16
NW = NC * NS
CH = 80
NCHUNK = 128
EPW = NCHUNK * CH
EP = NW * EPW
RPT = NP // NS
PAD_ROW = NP - 1

_sc_mesh = plsc.VectorSubcoreMesh(core_axis_name="c", subcore_axis_name="s")


@functools.partial(
    pl.kernel,
    mesh=_sc_mesh,
    out_type=jax.ShapeDtypeStruct((NC, NP, D), jnp.float32),
    scratch_types=[
        pltpu.VMEM((CH,), jnp.int32),
        pltpu.VMEM((CH,), jnp.int32),
        pltpu.VMEM((CH,), jnp.int32),
        pltpu.VMEM((CH,), jnp.int32),
        pltpu.VMEM((CH, D), jnp.float32),
        pltpu.VMEM((CH, D), jnp.float32),
        pltpu.VMEM_SHARED((NP, D), jnp.float32),
        pltpu.SemaphoreType.DMA,
        pltpu.SemaphoreType.DMA,
        pltpu.SemaphoreType.DMA,
        pltpu.SemaphoreType.DMA,
    ],
)
def _segsum_cell(src_hbm, dst_hbm, h_hbm, zeros_hbm, out_hbm,
                 src_a, dst_a, src_b, dst_b, rows_a, rows_b, acc_sh,
                 sia, sib, sra, srb):
    c = lax.axis_index("c")
    s = lax.axis_index("s")
    wid = s * NC + c
    base = wid * EPW

    def idx_off(ck):
        return pl.multiple_of(base + ck * CH, 8)

    def idx_start(ck, sv, dv, sem):
        off = idx_off(ck)
        pltpu.async_copy(src_hbm.at[pl.ds(off, CH)], sv, sem)
        pltpu.async_copy(dst_hbm.at[pl.ds(off, CH)], dv, sem)

    def idx_wait(ck, sv, dv, sem):
        off = idx_off(ck)
        pltpu.make_async_copy(src_hbm.at[pl.ds(off, CH)], sv, sem).wait()
        pltpu.make_async_copy(dst_hbm.at[pl.ds(off, CH)], dv, sem).wait()

    idx_start(0, src_a, dst_a, sia)
    idx_start(1, src_b, dst_b, sib)
    pltpu.sync_copy(zeros_hbm, acc_sh.at[pl.ds(s * RPT, RPT)])
    plsc.subcore_barrier()

    idx_wait(0, src_a, dst_a, sia)
    pltpu.async_copy(h_hbm.at[src_a], rows_a, sra)

    def body(j, carry):
        ca = 2 * j
        cb = 2 * j + 1
        idx_wait(cb, src_b, dst_b, sib)
        pltpu.async_copy(h_hbm.at[src_b], rows_b, srb)
        pltpu.make_async_copy(h_hbm.at[src_a], rows_a, sra).wait()
        pltpu.sync_copy(rows_a, acc_sh.at[dst_a], add=True)
        na = jnp.minimum(ca + 2, NCHUNK - 1)
        idx_start(na, src_a, dst_a, sia)
        pltpu.make_async_copy(h_hbm.at[src_b], rows_b, srb).wait()
        pltpu.sync_copy(rows_b, acc_sh.at[dst_b], add=True)
        idx_start(jnp.minimum(cb + 2, NCHUNK - 1), src_b, dst_b, sib)
        idx_wait(na, src_a, dst_a, sia)
        pltpu.async_copy(h_hbm.at[src_a], rows_a, sra)
        return carry

    lax.fori_loop(0, NCHUNK // 2, body, 0)
    pltpu.make_async_copy(h_hbm.at[src_a], rows_a, sra).wait()
    idx_wait(NCHUNK - 1, src_b, dst_b, sib)
    plsc.subcore_barrier()
    pltpu.sync_copy(acc_sh.at[pl.ds(s * RPT, RPT)],
                    out_hbm.at[c, pl.ds(s * RPT, RPT)])


def _gin_body(h_ref, p_ref, w1_ref, b1_ref, w2_ref, b2_ref, o_ref):
    z = h_ref[...] + p_ref[0] + p_ref[1]
    u = jnp.maximum(
        jnp.dot(z, w1_ref[...], preferred_element_type=jnp.float32) + b1_ref[...], 0.0)
    o_ref[...] = jnp.maximum(
        jnp.dot(u, w2_ref[...], preferred_element_type=jnp.float32) + b2_ref[...], 0.0)


def _gin_tc(h, p, W1, b1, W2, b2):
    RB = 2048
    return pl.pallas_call(
        _gin_body,
        grid=(NP // RB,),
        in_specs=[
            pl.BlockSpec((RB, D), lambda i: (i, 0)),
            pl.BlockSpec((NC, RB, D), lambda i: (0, i, 0)),
            pl.BlockSpec((D, D), lambda i: (0, 0)),
            pl.BlockSpec((1, D), lambda i: (0, 0)),
            pl.BlockSpec((D, D), lambda i: (0, 0)),
            pl.BlockSpec((1, D), lambda i: (0, 0)),
        ],
        out_specs=pl.BlockSpec((RB, D), lambda i: (i, 0)),
        out_shape=jax.ShapeDtypeStruct((NP, D), jnp.float32),
    )(h, p, W1, b1.reshape(1, D), W2, b2.reshape(1, D))


EB = 1024


def _at_body(s_ref, d_ref, o_ref):
    i = pl.program_id(0)
    se = s_ref[0, 0, :]
    de = d_ref[0, 0, :]
    cols = lax.broadcasted_iota(jnp.int32, (EB, N_TISSUE), 1)
    ohs = (se[:, None] == cols).astype(jnp.float32)
    ohd = (de[:, None] == cols).astype(jnp.float32)
    prod = lax.dot_general(ohd, ohs, (((0,), (0,)), ((), ())),
                           preferred_element_type=jnp.float32)

    @pl.when(i == 0)
    def _():
        o_ref[...] = prod

    @pl.when(i != 0)
    def _():
        o_ref[...] += prod


def _build_at(src3, dst3):
    return pl.pallas_call(
        _at_body,
        grid=(E_TISSUE // EB,),
        in_specs=[
            pl.BlockSpec((1, 1, EB), lambda i: (i, 0, 0)),
            pl.BlockSpec((1, 1, EB), lambda i: (i, 0, 0)),
        ],
        out_specs=pl.BlockSpec((N_TISSUE, N_TISSUE), lambda i: (0, 0)),
        out_shape=jax.ShapeDtypeStruct((N_TISSUE, N_TISSUE), jnp.float32),
    )(src3, dst3)


RB2 = 1000
NG2 = N_CELL // RB2


def _tail_body(h1_ref, h2_ref, h3_ref, a_ref, xt_ref, at_ref,
               t1w1, t1b1, t1w2, t1b2, t2w1, t2b1, t2w2, t2b2,
               t3w1, t3b1, t3w2, t3b2, cw1, cb1, cw2, cb2,
               o_ref, zc_ref):
    i = pl.program_id(0)
    a = a_ref[0, 0, :]
    z = jnp.concatenate([h1_ref[...], h2_ref[...], h3_ref[...]], axis=1)
    cols = lax.broadcasted_iota(jnp.int32, (RB2, N_TISSUE), 1)
    oh = (a[:, None] == cols).astype(jnp.float32)
    part = lax.dot_general(oh, z, (((0,), (0,)), ((), ())),
                           preferred_element_type=jnp.float32)

    @pl.when(i == 0)
    def _():
        zc_ref[...] = part

    @pl.when(i != 0)
    def _():
        zc_ref[...] += part

    @pl.when(i == NG2 - 1)
    def _():
        ht = jnp.concatenate([zc_ref[...], xt_ref[...]], axis=1)
        A = at_ref[...]
        touts = []
        for (w1, b1, w2, b2) in ((t1w1, t1b1, t1w2, t1b2),
                                 (t2w1, t2b1, t2w2, t2b2),
                                 (t3w1, t3b1, t3w2, t3b2)):
            agg = jnp.dot(A, ht, preferred_element_type=jnp.float32)
            zt = ht + agg
            u = jnp.maximum(
                jnp.dot(zt, w1[...], preferred_element_type=jnp.float32) + b1[...], 0.0)
            ht = jnp.maximum(
                jnp.dot(u, w2[...], preferred_element_type=jnp.float32) + b2[...], 0.0)
            touts.append(ht)
        ztc = jnp.concatenate(touts, axis=1)
        g = jnp.sum(ztc, axis=0, keepdims=True) * (1.0 / N_TISSUE)
        hc = jnp.maximum(
            jnp.dot(g, cw1[...], preferred_element_type=jnp.float32) + cb1[...], 0.0)
        o_ref[...] = jnp.dot(hc, cw2[...], preferred_element_type=jnp.float32) + cb2[...]


def _tail(h1, h2, h3, asg3, x_tissue, At, tw, cls_W1, cls_b1, cls_W2p, cls_b2p):
    def full(shape):
        nzero = len(shape)
        return pl.BlockSpec(shape, lambda i, _n=nzero: (0,) * _n)

    in_specs = [
        pl.BlockSpec((RB2, D), lambda i: (i, 0)),
        pl.BlockSpec((RB2, D), lambda i: (i, 0)),
        pl.BlockSpec((RB2, D), lambda i: (i, 0)),
        pl.BlockSpec((1, 1, RB2), lambda i: (i, 0, 0)),
        full((N_TISSUE, D)),
        full((N_TISSUE, N_TISSUE)),
    ]
    args = [h1, h2, h3, asg3, x_tissue, At]
    for (W1, b1, W2, b2) in tw:
        in_specs += [full(W1.shape), full((1, D)), full(W2.shape), full((1, D))]
        args += [W1, b1.reshape(1, D), W2, b2.reshape(1, D)]
    in_specs += [full(cls_W1.shape), full((1, D)),
                 full(cls_W2p.shape), full((1, D))]
    args += [cls_W1, cls_b1.reshape(1, D), cls_W2p, cls_b2p]
    return pl.pallas_call(
        _tail_body,
        grid=(NG2,),
        in_specs=in_specs,
        out_specs=pl.BlockSpec((1, D), lambda i: (0, 0)),
        out_shape=jax.ShapeDtypeStruct((1, D), jnp.float32),
        scratch_shapes=[pltpu.VMEM((N_TISSUE, 3 * D), jnp.float32)],
    )(*args)


def kernel(x_cell, x_tissue, edge_index_cell, edge_index_tissue, assignment,
           x_cell_batch, x_tissue_batch,
           c1_W1, c1_b1, c1_W2, c1_b2, c2_W1, c2_b1, c2_W2, c2_b2,
           c3_W1, c3_b1, c3_W2, c3_b2,
           t1_W1, t1_b1, t1_W2, t1_b2, t2_W1, t2_b1, t2_W2, t2_b2,
           t3_W1, t3_b1, t3_W2, t3_b2,
           cls_W1, cls_b1, cls_W2, cls_b2):
    npad = EP - E_CELL
    pad_rows = N_CELL + (jnp.arange(npad, dtype=jnp.int32) % (NP - N_CELL))
    src_c = jnp.concatenate([edge_index_cell[0], pad_rows])
    dst_c = jnp.concatenate([edge_index_cell[1], pad_rows])
    zeros = jnp.zeros((RPT, D), jnp.float32)

    h = jnp.zeros((NP, D), jnp.float32).at[:N_CELL].set(x_cell)
    hs = []
    for (W1, b1, W2, b2) in ((c1_W1, c1_b1, c1_W2, c1_b2),
                             (c2_W1, c2_b1, c2_W2, c2_b2),
                             (c3_W1, c3_b1, c3_W2, c3_b2)):
        p = _segsum_cell(src_c, dst_c, h, zeros)
        h = _gin_tc(h, p, W1, b1, W2, b2)
        hs.append(h)

    At = _build_at(edge_index_tissue[0].reshape(E_TISSUE // EB, 1, EB),
                   edge_index_tissue[1].reshape(E_TISSUE // EB, 1, EB))

    cls_W2p = jnp.zeros((D, D), jnp.float32).at[:, :7].set(cls_W2)
    cls_b2p = jnp.zeros((1, D), jnp.float32).at[:, :7].set(cls_b2)

    out = _tail(hs[0], hs[1], hs[2], assignment.reshape(NG2, 1, RB2),
                x_tissue, At,
                ((t1_W1, t1_b1, t1_W2, t1_b2),
                 (t2_W1, t2_b1, t2_W2, t2_b2),
                 (t3_W1, t3_b1, t3_W2, t3_b2)),
                cls_W1, cls_b1, cls_W2p, cls_b2p)
    return out[:, :7]


# speedup vs baseline: 3.9363x; 1.0522x over previous
import functools

import jax
import jax.numpy as jnp
from jax import lax
from jax.experimental import pallas as pl
from jax.experimental.pallas import tpu as pltpu
from jax.experimental.pallas import tpu_sc as plsc

N_CELL, N_TISSUE, D = 10000, 512, 128
NP = 10240
E_CELL, E_TISSUE = 320000, 4096
NC, NS = 2, 16
NW = NC * NS
CH = 120
NCHUNK = 86
EPW = NCHUNK * CH
EP = NW * EPW
RPT = NP // NS
PAD_ROW = NP - 1

_sc_mesh = plsc.VectorSubcoreMesh(core_axis_name="c", subcore_axis_name="s")


@functools.partial(
    pl.kernel,
    mesh=_sc_mesh,
    out_type=jax.ShapeDtypeStruct((NC, NP, D), jnp.float32),
    scratch_types=[
        pltpu.VMEM((CH,), jnp.int32),
        pltpu.VMEM((CH,), jnp.int32),
        pltpu.VMEM((CH,), jnp.int32),
        pltpu.VMEM((CH,), jnp.int32),
        pltpu.VMEM((CH, D), jnp.float32),
        pltpu.VMEM((CH, D), jnp.float32),
        pltpu.VMEM_SHARED((NP, D), jnp.float32),
        pltpu.SemaphoreType.DMA,
        pltpu.SemaphoreType.DMA,
        pltpu.SemaphoreType.DMA,
        pltpu.SemaphoreType.DMA,
    ],
)
def _segsum_cell(src_hbm, dst_hbm, h_hbm, zeros_hbm, out_hbm,
                 src_a, dst_a, src_b, dst_b, rows_a, rows_b, acc_sh,
                 sia, sib, sra, srb):
    c = lax.axis_index("c")
    s = lax.axis_index("s")
    wid = s * NC + c
    base = wid * EPW

    def idx_off(ck):
        return pl.multiple_of(base + ck * CH, 8)

    def idx_start(ck, sv, dv, sem):
        off = idx_off(ck)
        pltpu.async_copy(src_hbm.at[pl.ds(off, CH)], sv, sem)
        pltpu.async_copy(dst_hbm.at[pl.ds(off, CH)], dv, sem)

    def idx_wait(ck, sv, dv, sem):
        off = idx_off(ck)
        pltpu.make_async_copy(src_hbm.at[pl.ds(off, CH)], sv, sem).wait()
        pltpu.make_async_copy(dst_hbm.at[pl.ds(off, CH)], dv, sem).wait()

    idx_start(0, src_a, dst_a, sia)
    idx_start(1, src_b, dst_b, sib)
    pltpu.sync_copy(zeros_hbm, acc_sh.at[pl.ds(s * RPT, RPT)])
    plsc.subcore_barrier()

    idx_wait(0, src_a, dst_a, sia)
    pltpu.async_copy(h_hbm.at[src_a], rows_a, sra)

    def body(j, carry):
        ca = 2 * j
        cb = 2 * j + 1
        idx_wait(cb, src_b, dst_b, sib)
        pltpu.async_copy(h_hbm.at[src_b], rows_b, srb)
        pltpu.make_async_copy(h_hbm.at[src_a], rows_a, sra).wait()
        pltpu.sync_copy(rows_a, acc_sh.at[dst_a], add=True)
        na = jnp.minimum(ca + 2, NCHUNK - 1)
        idx_start(na, src_a, dst_a, sia)
        pltpu.make_async_copy(h_hbm.at[src_b], rows_b, srb).wait()
        pltpu.sync_copy(rows_b, acc_sh.at[dst_b], add=True)
        idx_start(jnp.minimum(cb + 2, NCHUNK - 1), src_b, dst_b, sib)
        idx_wait(na, src_a, dst_a, sia)
        pltpu.async_copy(h_hbm.at[src_a], rows_a, sra)
        return carry

    lax.fori_loop(0, NCHUNK // 2, body, 0)
    pltpu.make_async_copy(h_hbm.at[src_a], rows_a, sra).wait()
    idx_wait(NCHUNK - 1, src_b, dst_b, sib)
    plsc.subcore_barrier()
    pltpu.sync_copy(acc_sh.at[pl.ds(s * RPT, RPT)],
                    out_hbm.at[c, pl.ds(s * RPT, RPT)])


def _gin_body(h_ref, p_ref, w1_ref, b1_ref, w2_ref, b2_ref, o_ref):
    z = h_ref[...] + p_ref[0] + p_ref[1]
    u = jnp.maximum(
        jnp.dot(z, w1_ref[...], preferred_element_type=jnp.float32) + b1_ref[...], 0.0)
    o_ref[...] = jnp.maximum(
        jnp.dot(u, w2_ref[...], preferred_element_type=jnp.float32) + b2_ref[...], 0.0)


def _gin_tc(h, p, W1, b1, W2, b2):
    RB = 2048
    return pl.pallas_call(
        _gin_body,
        grid=(NP // RB,),
        in_specs=[
            pl.BlockSpec((RB, D), lambda i: (i, 0)),
            pl.BlockSpec((NC, RB, D), lambda i: (0, i, 0)),
            pl.BlockSpec((D, D), lambda i: (0, 0)),
            pl.BlockSpec((1, D), lambda i: (0, 0)),
            pl.BlockSpec((D, D), lambda i: (0, 0)),
            pl.BlockSpec((1, D), lambda i: (0, 0)),
        ],
        out_specs=pl.BlockSpec((RB, D), lambda i: (i, 0)),
        out_shape=jax.ShapeDtypeStruct((NP, D), jnp.float32),
    )(h, p, W1, b1.reshape(1, D), W2, b2.reshape(1, D))


EB = 1024


def _at_body(s_ref, d_ref, o_ref):
    i = pl.program_id(0)
    se = s_ref[0, 0, :]
    de = d_ref[0, 0, :]
    cols = lax.broadcasted_iota(jnp.int32, (EB, N_TISSUE), 1)
    ohs = (se[:, None] == cols).astype(jnp.float32)
    ohd = (de[:, None] == cols).astype(jnp.float32)
    prod = lax.dot_general(ohd, ohs, (((0,), (0,)), ((), ())),
                           preferred_element_type=jnp.float32)

    @pl.when(i == 0)
    def _():
        o_ref[...] = prod

    @pl.when(i != 0)
    def _():
        o_ref[...] += prod


def _build_at(src3, dst3):
    return pl.pallas_call(
        _at_body,
        grid=(E_TISSUE // EB,),
        in_specs=[
            pl.BlockSpec((1, 1, EB), lambda i: (i, 0, 0)),
            pl.BlockSpec((1, 1, EB), lambda i: (i, 0, 0)),
        ],
        out_specs=pl.BlockSpec((N_TISSUE, N_TISSUE), lambda i: (0, 0)),
        out_shape=jax.ShapeDtypeStruct((N_TISSUE, N_TISSUE), jnp.float32),
    )(src3, dst3)


RB2 = 1000
NG2 = N_CELL // RB2


def _tail_body(h1_ref, h2_ref, h3_ref, a_ref, xt_ref, at_ref,
               t1w1, t1b1, t1w2, t1b2, t2w1, t2b1, t2w2, t2b2,
               t3w1, t3b1, t3w2, t3b2, cw1, cb1, cw2, cb2,
               o_ref, zc_ref):
    i = pl.program_id(0)
    a = a_ref[0, 0, :]
    z = jnp.concatenate([h1_ref[...], h2_ref[...], h3_ref[...]], axis=1)
    cols = lax.broadcasted_iota(jnp.int32, (RB2, N_TISSUE), 1)
    oh = (a[:, None] == cols).astype(jnp.float32)
    part = lax.dot_general(oh, z, (((0,), (0,)), ((), ())),
                           preferred_element_type=jnp.float32)

    @pl.when(i == 0)
    def _():
        zc_ref[...] = part

    @pl.when(i != 0)
    def _():
        zc_ref[...] += part

    @pl.when(i == NG2 - 1)
    def _():
        ht = jnp.concatenate([zc_ref[...], xt_ref[...]], axis=1)
        A = at_ref[...]
        touts = []
        for (w1, b1, w2, b2) in ((t1w1, t1b1, t1w2, t1b2),
                                 (t2w1, t2b1, t2w2, t2b2),
                                 (t3w1, t3b1, t3w2, t3b2)):
            agg = jnp.dot(A, ht, preferred_element_type=jnp.float32)
            zt = ht + agg
            u = jnp.maximum(
                jnp.dot(zt, w1[...], preferred_element_type=jnp.float32) + b1[...], 0.0)
            ht = jnp.maximum(
                jnp.dot(u, w2[...], preferred_element_type=jnp.float32) + b2[...], 0.0)
            touts.append(ht)
        ztc = jnp.concatenate(touts, axis=1)
        g = jnp.sum(ztc, axis=0, keepdims=True) * (1.0 / N_TISSUE)
        hc = jnp.maximum(
            jnp.dot(g, cw1[...], preferred_element_type=jnp.float32) + cb1[...], 0.0)
        o_ref[...] = jnp.dot(hc, cw2[...], preferred_element_type=jnp.float32) + cb2[...]


def _tail(h1, h2, h3, asg3, x_tissue, At, tw, cls_W1, cls_b1, cls_W2p, cls_b2p):
    def full(shape):
        nzero = len(shape)
        return pl.BlockSpec(shape, lambda i, _n=nzero: (0,) * _n)

    in_specs = [
        pl.BlockSpec((RB2, D), lambda i: (i, 0)),
        pl.BlockSpec((RB2, D), lambda i: (i, 0)),
        pl.BlockSpec((RB2, D), lambda i: (i, 0)),
        pl.BlockSpec((1, 1, RB2), lambda i: (i, 0, 0)),
        full((N_TISSUE, D)),
        full((N_TISSUE, N_TISSUE)),
    ]
    args = [h1, h2, h3, asg3, x_tissue, At]
    for (W1, b1, W2, b2) in tw:
        in_specs += [full(W1.shape), full((1, D)), full(W2.shape), full((1, D))]
        args += [W1, b1.reshape(1, D), W2, b2.reshape(1, D)]
    in_specs += [full(cls_W1.shape), full((1, D)),
                 full(cls_W2p.shape), full((1, D))]
    args += [cls_W1, cls_b1.reshape(1, D), cls_W2p, cls_b2p]
    return pl.pallas_call(
        _tail_body,
        grid=(NG2,),
        in_specs=in_specs,
        out_specs=pl.BlockSpec((1, D), lambda i: (0, 0)),
        out_shape=jax.ShapeDtypeStruct((1, D), jnp.float32),
        scratch_shapes=[pltpu.VMEM((N_TISSUE, 3 * D), jnp.float32)],
    )(*args)


def kernel(x_cell, x_tissue, edge_index_cell, edge_index_tissue, assignment,
           x_cell_batch, x_tissue_batch,
           c1_W1, c1_b1, c1_W2, c1_b2, c2_W1, c2_b1, c2_W2, c2_b2,
           c3_W1, c3_b1, c3_W2, c3_b2,
           t1_W1, t1_b1, t1_W2, t1_b2, t2_W1, t2_b1, t2_W2, t2_b2,
           t3_W1, t3_b1, t3_W2, t3_b2,
           cls_W1, cls_b1, cls_W2, cls_b2):
    npad = EP - E_CELL
    pad_rows = N_CELL + (jnp.arange(npad, dtype=jnp.int32) % (NP - N_CELL))
    src_c = jnp.concatenate([edge_index_cell[0], pad_rows])
    dst_c = jnp.concatenate([edge_index_cell[1], pad_rows])
    zeros = jnp.zeros((RPT, D), jnp.float32)

    h = jnp.zeros((NP, D), jnp.float32).at[:N_CELL].set(x_cell)
    hs = []
    for (W1, b1, W2, b2) in ((c1_W1, c1_b1, c1_W2, c1_b2),
                             (c2_W1, c2_b1, c2_W2, c2_b2),
                             (c3_W1, c3_b1, c3_W2, c3_b2)):
        p = _segsum_cell(src_c, dst_c, h, zeros)
        h = _gin_tc(h, p, W1, b1, W2, b2)
        hs.append(h)

    At = _build_at(edge_index_tissue[0].reshape(E_TISSUE // EB, 1, EB),
                   edge_index_tissue[1].reshape(E_TISSUE // EB, 1, EB))

    cls_W2p = jnp.zeros((D, D), jnp.float32).at[:, :7].set(cls_W2)
    cls_b2p = jnp.zeros((1, D), jnp.float32).at[:, :7].set(cls_b2)

    out = _tail(hs[0], hs[1], hs[2], assignment.reshape(NG2, 1, RB2),
                x_tissue, At,
                ((t1_W1, t1_b1, t1_W2, t1_b2),
                 (t2_W1, t2_b1, t2_W2, t2_b2),
                 (t3_W1, t3_b1, t3_W2, t3_b2)),
                cls_W1, cls_b1, cls_W2p, cls_b2p)
    return out[:, :7]
